# baseline (speedup 1.0000x reference)
import functools

import jax
import jax.numpy as jnp
from jax.experimental import pallas as pl
from jax.experimental.pallas import tpu as pltpu

_S8 = [(0, 1), (2, 3), (0, 2), (1, 3), (1, 2), (4, 5), (6, 7), (4, 6),
       (5, 7), (5, 6), (0, 4), (2, 6), (2, 4), (1, 5), (3, 7), (3, 5),
       (1, 2), (3, 4), (5, 6)]
_TOP8_PAIRS = (
    _S8
    + [(i + 8, j + 8) for (i, j) in _S8]
    + [(i, 15 - i) for i in range(8)]
    + [(i, i + 4) for i in range(4)]
    + [(i, i + 2) for i in (0, 1, 4, 5)]
    + [(i, i + 1) for i in (0, 2, 4, 6)]
)


def _stage_body(nbr_ref, self_ref, wa_ref, ba_ref, wb_ref, bb_ref,
                wr_ref, br_ref, out_ref, *, d, add_self_res):
    x = nbr_ref[...]
    vals = [x[:, j * d:(j + 1) * d] for j in range(16)]
    for i, j in _TOP8_PAIRS:
        a, b = vals[i], vals[j]
        vals[i] = jnp.maximum(a, b)
        vals[j] = jnp.minimum(a, b)
    s = self_ref[...]
    sh = s.astype(jnp.bfloat16)
    seq = [sh] + [v.astype(jnp.bfloat16) for v in vals[:8]]

    wa = wa_ref[...]
    hs = []
    for t in range(5):
        win = jnp.concatenate(seq[t:t + 5], axis=1)
        h = jnp.dot(win, wa, preferred_element_type=jnp.float32)
        hs.append(jnp.maximum(h + ba_ref[...], 0.0).astype(jnp.bfloat16))
    big = jnp.concatenate(hs, axis=1)
    o = jnp.dot(big, wb_ref[...], preferred_element_type=jnp.float32)
    o = jnp.maximum(o + bb_ref[...], 0.0)
    if add_self_res:
        o = o + s
    else:
        o = o + jnp.dot(sh, wr_ref[...],
                        preferred_element_type=jnp.float32) + br_ref[...]
    out_ref[...] = o


def _stage(nbr, selfx, WA, bA, WB, bB, Wres, bres, *, add_self_res,
           block_n=512):
    n, d = selfx.shape
    mid = WA.shape[0]
    out_dim = WB.shape[0]
    bn = min(block_n, n)
    wa = jnp.transpose(WA, (2, 1, 0)).reshape(5 * d, mid).astype(jnp.bfloat16)
    wb = jnp.transpose(WB, (2, 1, 0)).reshape(5 * mid, out_dim).astype(
        jnp.bfloat16)
    if add_self_res:
        wr = jnp.zeros((8, 128), jnp.bfloat16)
        br = jnp.zeros((1, 128), jnp.float32)
    else:
        wr = Wres.T.astype(jnp.bfloat16)
        br = bres.reshape(1, out_dim)
    body = functools.partial(_stage_body, d=d, add_self_res=add_self_res)
    rep = lambda i: (0, 0)
    return pl.pallas_call(
        body,
        grid=(n // bn,),
        in_specs=[
            pl.BlockSpec((bn, 16 * d), lambda i: (i, 0)),
            pl.BlockSpec((bn, d), lambda i: (i, 0)),
            pl.BlockSpec(wa.shape, rep),
            pl.BlockSpec((1, mid), rep),
            pl.BlockSpec(wb.shape, rep),
            pl.BlockSpec((1, out_dim), rep),
            pl.BlockSpec(wr.shape, rep),
            pl.BlockSpec(br.shape, rep),
        ],
        out_specs=pl.BlockSpec((bn, out_dim), lambda i: (i, 0)),
        out_shape=jax.ShapeDtypeStruct((n, out_dim), jnp.float32),
    )(nbr, selfx, wa, bA.reshape(1, mid), wb, bB.reshape(1, out_dim),
      wr, br)


def kernel(sample_0, sample_1, sample_2, W000, b000, W001, b001, W010,
           b010, W011, b011, Wres0, bres0, W100, b100, W101, b101):
    b, d_in = sample_0.shape
    f0 = sample_1.shape[0] // b
    h1 = _stage(sample_2.reshape(b * f0, -1), sample_1,
                W010, b010, W011, b011, Wres0, bres0, add_self_res=False)
    h0 = _stage(sample_1.reshape(b, -1), sample_0,
                W000, b000, W001, b001, Wres0, bres0, add_self_res=False)
    out = _stage(h1.reshape(b, -1), h0,
                 W100, b100, W101, b101, None, None, add_self_res=True)
    return out


# speedup vs baseline: 1.0328x; 1.0328x over previous
import functools

import jax
import jax.numpy as jnp
from jax.experimental import pallas as pl
from jax.experimental.pallas import tpu as pltpu

_S8 = [(0, 1), (2, 3), (0, 2), (1, 3), (1, 2), (4, 5), (6, 7), (4, 6),
       (5, 7), (5, 6), (0, 4), (2, 6), (2, 4), (1, 5), (3, 7), (3, 5),
       (1, 2), (3, 4), (5, 6)]
_TOP8_PAIRS = (
    _S8
    + [(i + 8, j + 8) for (i, j) in _S8]
    + [(i, 15 - i) for i in range(8)]
    + [(i, i + 4) for i in range(4)]
    + [(i, i + 2) for i in (0, 1, 4, 5)]
    + [(i, i + 1) for i in (0, 2, 4, 6)]
)


def _stage_body(nbr_ref, self_ref, wa_ref, ba_ref, wb_ref, bb_ref,
                wr_ref, br_ref, out_ref, *, d, add_self_res):
    x = nbr_ref[...]
    vals = [x[:, j * d:(j + 1) * d].astype(jnp.bfloat16) for j in range(16)]
    for i, j in _TOP8_PAIRS:
        a, b = vals[i], vals[j]
        vals[i] = jnp.maximum(a, b)
        vals[j] = jnp.minimum(a, b)
    s = self_ref[...]
    sh = s.astype(jnp.bfloat16)
    seq = [sh] + vals[:8]

    wa = wa_ref[...]
    accs = [None] * 5
    for k in range(5):
        wak = wa[k * d:(k + 1) * d, :]
        for t in range(5):
            p = jnp.dot(seq[t + k], wak, preferred_element_type=jnp.float32)
            accs[t] = p if accs[t] is None else accs[t] + p
    hs = [jnp.maximum(a + ba_ref[...], 0.0).astype(jnp.bfloat16)
          for a in accs]
    wb = wb_ref[...]
    mid = wb.shape[0] // 5
    o = None
    for t in range(5):
        p = jnp.dot(hs[t], wb[t * mid:(t + 1) * mid, :],
                    preferred_element_type=jnp.float32)
        o = p if o is None else o + p
    o = jnp.maximum(o + bb_ref[...], 0.0)
    if add_self_res:
        o = o + s
    else:
        o = o + jnp.dot(sh, wr_ref[...],
                        preferred_element_type=jnp.float32) + br_ref[...]
    out_ref[...] = o


def _stage(nbr, selfx, WA, bA, WB, bB, Wres, bres, *, add_self_res,
           block_n=512):
    n, d = selfx.shape
    mid = WA.shape[0]
    out_dim = WB.shape[0]
    bn = min(block_n, n)
    wa = jnp.transpose(WA, (2, 1, 0)).reshape(5 * d, mid).astype(jnp.bfloat16)
    wb = jnp.transpose(WB, (2, 1, 0)).reshape(5 * mid, out_dim).astype(
        jnp.bfloat16)
    if add_self_res:
        wr = jnp.zeros((8, 128), jnp.bfloat16)
        br = jnp.zeros((1, 128), jnp.float32)
    else:
        wr = Wres.T.astype(jnp.bfloat16)
        br = bres.reshape(1, out_dim)
    body = functools.partial(_stage_body, d=d, add_self_res=add_self_res)
    rep = lambda i: (0, 0)
    return pl.pallas_call(
        body,
        grid=(n // bn,),
        in_specs=[
            pl.BlockSpec((bn, 16 * d), lambda i: (i, 0)),
            pl.BlockSpec((bn, d), lambda i: (i, 0)),
            pl.BlockSpec(wa.shape, rep),
            pl.BlockSpec((1, mid), rep),
            pl.BlockSpec(wb.shape, rep),
            pl.BlockSpec((1, out_dim), rep),
            pl.BlockSpec(wr.shape, rep),
            pl.BlockSpec(br.shape, rep),
        ],
        out_specs=pl.BlockSpec((bn, out_dim), lambda i: (i, 0)),
        out_shape=jax.ShapeDtypeStruct((n, out_dim), jnp.float32),
    )(nbr, selfx, wa, bA.reshape(1, mid), wb, bB.reshape(1, out_dim),
      wr, br)


def kernel(sample_0, sample_1, sample_2, W000, b000, W001, b001, W010,
           b010, W011, b011, Wres0, bres0, W100, b100, W101, b101):
    b, d_in = sample_0.shape
    f0 = sample_1.shape[0] // b
    h1 = _stage(sample_2.reshape(b * f0, -1), sample_1,
                W010, b010, W011, b011, Wres0, bres0, add_self_res=False)
    h0 = _stage(sample_1.reshape(b, -1), sample_0,
                W000, b000, W001, b001, Wres0, bres0, add_self_res=False)
    out = _stage(h1.reshape(b, -1), h0,
                 W100, b100, W101, b101, None, None, add_self_res=True)
    return out
